# baseline (speedup 1.0000x reference)
import functools

import jax
import jax.numpy as jnp
from jax import lax
from jax.experimental import pallas as pl
from jax.experimental.pallas import tpu as pltpu
from jax.experimental.pallas import tpu_sc as plsc

B = 16384
D = 32
NSK = 8
NUM_SKILL_COLS = 65
OUT_D = D + 4 + NUM_SKILL_COLS
OUT_P = 128
V = 100001
NW = 32
BPW = B // NW
AGE_MEAN = 45.0
AGE_STD = 20.0

_NB = 196
_S = _NB * OUT_P


def _k1_body(t0, t1, t2, t3, out):
    x = jnp.concatenate([t0[...], t1[...], t2[...], t3[...]], axis=0)
    out[...] = x.T


_k1 = pl.pallas_call(
    _k1_body,
    grid=(_NB,),
    in_specs=[
        pl.BlockSpec((D, OUT_P), lambda i, k=k: (0, i + k * _NB))
        for k in range(4)
    ],
    out_specs=pl.BlockSpec((OUT_P, OUT_P), lambda i: (i, 0)),
    out_shape=jax.ShapeDtypeStruct((_S, OUT_P), jnp.float32),
)


def _body(table_hbm, pid_hbm, age_hbm, freq_hbm, lat_hbm, lon_hbm,
          skills_hbm, out_hbm, idx_v, rows_v, skills_v, age_v, freq_v,
          lat_v, lon_v, slab, sem):
    c = lax.axis_index("c")
    s = lax.axis_index("s")
    wid = s * 2 + c
    base = wid * BPW

    pltpu.sync_copy(pid_hbm.at[pl.ds(base, BPW)], idx_v)

    one = jnp.ones((16,), jnp.int32)
    nil = jnp.zeros((16,), jnp.int32)

    @plsc.parallel_loop(0, BPW // 16, unroll=4)
    def _remap(i):
        r = idx_v[pl.ds(i * 16, 16)]
        a = (jnp.where(r >= _S, one, nil) + jnp.where(r >= 2 * _S, one, nil)
             + jnp.where(r >= 3 * _S, one, nil))
        idx_v[pl.ds(i * 16, 16)] = (r - a * _S) * 4 + a

    gather = pltpu.async_copy(table_hbm.at[idx_v], rows_v, sem)
    pltpu.sync_copy(skills_hbm.at[pl.ds(base * NSK, BPW * NSK)], skills_v)
    pltpu.sync_copy(age_hbm.at[pl.ds(base, BPW)], age_v)
    pltpu.sync_copy(freq_hbm.at[pl.ds(base, BPW)], freq_v)
    pltpu.sync_copy(lat_hbm.at[pl.ds(base, BPW)], lat_v)
    pltpu.sync_copy(lon_hbm.at[pl.ds(base, BPW)], lon_v)

    iota = lax.iota(jnp.int32, 16)
    zeros = jnp.zeros((16,), jnp.float32)
    ones = jnp.ones((16,), jnp.float32)

    @plsc.parallel_loop(0, BPW, unroll=8)
    def _zero(r):
        o = r * OUT_P
        slab[pl.ds(o + 36, 16)] = zeros
        slab[pl.ds(o + 52, 16)] = zeros
        slab[pl.ds(o + 68, 16)] = zeros
        slab[pl.ds(o + 84, 16)] = zeros
        slab[pl.ds(o + 85, 16)] = zeros

    rowoff = lax.shift_right_logical(iota, 3) * OUT_P + 36

    @plsc.parallel_loop(0, (BPW * NSK) // 16, unroll=8)
    def _mh(i):
        sk = skills_v[pl.ds(i * 16, 16)]
        flat = i * (2 * OUT_P) + rowoff + sk
        plsc.store_scatter(slab, [flat], ones)

    iota_row = iota * OUT_P

    @plsc.parallel_loop(0, BPW // 16, unroll=4)
    def _scal(i):
        b16 = i * 16
        flat = b16 * OUT_P + iota_row + D
        a = (age_v[pl.ds(b16, 16)] - AGE_MEAN) / AGE_STD
        plsc.store_scatter(slab, [flat], a)
        plsc.store_scatter(slab, [flat + 1], freq_v[pl.ds(b16, 16)])
        plsc.store_scatter(slab, [flat + 2], lat_v[pl.ds(b16, 16)])
        plsc.store_scatter(slab, [flat + 3], lon_v[pl.ds(b16, 16)])

    gather.wait()

    @plsc.parallel_loop(0, BPW, unroll=8)
    def _emb(r):
        o = r * OUT_P
        slab[pl.ds(o, 16)] = rows_v[r, pl.ds(0, 16)]
        slab[pl.ds(o + 16, 16)] = rows_v[r, pl.ds(16, 16)]

    pltpu.sync_copy(slab, out_hbm.at[pl.ds(base * OUT_P, BPW * OUT_P)])


_patient_sc = functools.partial(
    pl.kernel,
    out_type=jax.ShapeDtypeStruct((B * OUT_P,), jnp.float32),
    mesh=plsc.VectorSubcoreMesh(core_axis_name="c", subcore_axis_name="s"),
    compiler_params=pltpu.CompilerParams(
        needs_layout_passes=False, use_tc_tiling_on_sc=False),
    scratch_types=[
        pltpu.VMEM((BPW,), jnp.int32),
        pltpu.VMEM((BPW, D), jnp.float32),
        pltpu.VMEM((BPW * NSK,), jnp.int32),
        pltpu.VMEM((BPW,), jnp.float32),
        pltpu.VMEM((BPW,), jnp.float32),
        pltpu.VMEM((BPW,), jnp.float32),
        pltpu.VMEM((BPW,), jnp.float32),
        pltpu.VMEM((BPW * OUT_P,), jnp.float32),
        pltpu.SemaphoreType.DMA,
    ],
)(_body)

_K3_RB = 512


def _k3_body(fin, out):
    x = fin[...]
    out[...] = x.T[:OUT_D, :]


_k3 = pl.pallas_call(
    _k3_body,
    grid=(B // _K3_RB,),
    in_specs=[pl.BlockSpec((_K3_RB, OUT_P), lambda i: (i, 0))],
    out_specs=pl.BlockSpec((OUT_D, _K3_RB), lambda i: (0, i)),
    out_shape=jax.ShapeDtypeStruct((OUT_D, B), jnp.float32),
)


@jax.jit
def kernel(patient_id, patient_age, patient_dialysis_freq,
           patient_dialysis_latitude, patient_dialysis_longitude,
           patient_skills, emb_table):
    pid = patient_id.astype(jnp.int32)
    skills_flat = patient_skills.astype(jnp.int32).reshape(-1)
    tt = emb_table.T
    table_lin = _k1(tt, tt, tt, tt).reshape(4 * _S, D)
    flat = _patient_sc(table_lin, pid, patient_age, patient_dialysis_freq,
                       patient_dialysis_latitude,
                       patient_dialysis_longitude, skills_flat)
    return _k3(flat.reshape(B, OUT_P)).T


# speedup vs baseline: 1.9653x; 1.9653x over previous
import functools

import jax
import jax.numpy as jnp
from jax import lax
from jax.experimental import pallas as pl
from jax.experimental.pallas import tpu as pltpu
from jax.experimental.pallas import tpu_sc as plsc

B = 16384
D = 32
NSK = 8
NUM_SKILL_COLS = 65
OUT_D = D + 4 + NUM_SKILL_COLS
OUT_P = 128
V = 100001
NW = 32
BPW = B // NW
AGE_MEAN = 45.0
AGE_STD = 20.0

_K1_C = 512
_NB = 49
_S = _NB * _K1_C


def _k1_body(t0, t1, t2, t3, out):
    x = jnp.concatenate([t0[...], t1[...], t2[...], t3[...]], axis=0)
    out[...] = x.T


_k1 = pl.pallas_call(
    _k1_body,
    grid=(_NB,),
    in_specs=[
        pl.BlockSpec((D, _K1_C), lambda i, k=k: (0, i + k * _NB))
        for k in range(4)
    ],
    out_specs=pl.BlockSpec((_K1_C, OUT_P), lambda i: (i, 0)),
    out_shape=jax.ShapeDtypeStruct((_S, OUT_P), jnp.float32),
)


def _body(table_hbm, pid_hbm, age_hbm, freq_hbm, lat_hbm, lon_hbm,
          skills_hbm, out_hbm, idx_v, rows_v, skills_v, age_v, freq_v,
          lat_v, lon_v, slab, sem):
    c = lax.axis_index("c")
    s = lax.axis_index("s")
    wid = s * 2 + c
    base = wid * BPW

    pltpu.sync_copy(pid_hbm.at[pl.ds(base, BPW)], idx_v)

    one = jnp.ones((16,), jnp.int32)
    nil = jnp.zeros((16,), jnp.int32)

    @plsc.parallel_loop(0, BPW // 16, unroll=4)
    def _remap(i):
        r = idx_v[pl.ds(i * 16, 16)]
        a = (jnp.where(r >= _S, one, nil) + jnp.where(r >= 2 * _S, one, nil)
             + jnp.where(r >= 3 * _S, one, nil))
        idx_v[pl.ds(i * 16, 16)] = (r - a * _S) * 4 + a

    gather = pltpu.async_copy(table_hbm.at[idx_v], rows_v, sem)
    pltpu.sync_copy(skills_hbm.at[pl.ds(base * NSK, BPW * NSK)], skills_v)
    pltpu.sync_copy(age_hbm.at[pl.ds(base, BPW)], age_v)
    pltpu.sync_copy(freq_hbm.at[pl.ds(base, BPW)], freq_v)
    pltpu.sync_copy(lat_hbm.at[pl.ds(base, BPW)], lat_v)
    pltpu.sync_copy(lon_hbm.at[pl.ds(base, BPW)], lon_v)

    iota = lax.iota(jnp.int32, 16)
    zeros = jnp.zeros((16,), jnp.float32)
    ones = jnp.ones((16,), jnp.float32)

    @plsc.parallel_loop(0, BPW, unroll=8)
    def _zero(r):
        o = r * OUT_P
        slab[pl.ds(o + 36, 16)] = zeros
        slab[pl.ds(o + 52, 16)] = zeros
        slab[pl.ds(o + 68, 16)] = zeros
        slab[pl.ds(o + 84, 16)] = zeros
        slab[pl.ds(o + 85, 16)] = zeros

    rowoff = lax.shift_right_logical(iota, 3) * OUT_P + 36

    @plsc.parallel_loop(0, (BPW * NSK) // 16, unroll=8)
    def _mh(i):
        sk = skills_v[pl.ds(i * 16, 16)]
        flat = i * (2 * OUT_P) + rowoff + sk
        plsc.store_scatter(slab, [flat], ones)

    iota_row = iota * OUT_P

    @plsc.parallel_loop(0, BPW // 16, unroll=4)
    def _scal(i):
        b16 = i * 16
        flat = b16 * OUT_P + iota_row + D
        a = (age_v[pl.ds(b16, 16)] - AGE_MEAN) / AGE_STD
        plsc.store_scatter(slab, [flat], a)
        plsc.store_scatter(slab, [flat + 1], freq_v[pl.ds(b16, 16)])
        plsc.store_scatter(slab, [flat + 2], lat_v[pl.ds(b16, 16)])
        plsc.store_scatter(slab, [flat + 3], lon_v[pl.ds(b16, 16)])

    gather.wait()

    @plsc.parallel_loop(0, BPW, unroll=8)
    def _emb(r):
        o = r * OUT_P
        slab[pl.ds(o, 16)] = rows_v[r, pl.ds(0, 16)]
        slab[pl.ds(o + 16, 16)] = rows_v[r, pl.ds(16, 16)]

    pltpu.sync_copy(slab, out_hbm.at[pl.ds(base * OUT_P, BPW * OUT_P)])


_patient_sc = functools.partial(
    pl.kernel,
    out_type=jax.ShapeDtypeStruct((B * OUT_P,), jnp.float32),
    mesh=plsc.VectorSubcoreMesh(core_axis_name="c", subcore_axis_name="s"),
    compiler_params=pltpu.CompilerParams(
        needs_layout_passes=False, use_tc_tiling_on_sc=False),
    scratch_types=[
        pltpu.VMEM((BPW,), jnp.int32),
        pltpu.VMEM((BPW, D), jnp.float32),
        pltpu.VMEM((BPW * NSK,), jnp.int32),
        pltpu.VMEM((BPW,), jnp.float32),
        pltpu.VMEM((BPW,), jnp.float32),
        pltpu.VMEM((BPW,), jnp.float32),
        pltpu.VMEM((BPW,), jnp.float32),
        pltpu.VMEM((BPW * OUT_P,), jnp.float32),
        pltpu.SemaphoreType.DMA,
    ],
)(_body)

_K3_RB = 1024


def _k3_body(fin, out):
    x = fin[...]
    out[...] = x.T[:OUT_D, :]


_k3 = pl.pallas_call(
    _k3_body,
    grid=(B // _K3_RB,),
    in_specs=[pl.BlockSpec((_K3_RB, OUT_P), lambda i: (i, 0))],
    out_specs=pl.BlockSpec((OUT_D, _K3_RB), lambda i: (0, i)),
    out_shape=jax.ShapeDtypeStruct((OUT_D, B), jnp.float32),
)


@jax.jit
def kernel(patient_id, patient_age, patient_dialysis_freq,
           patient_dialysis_latitude, patient_dialysis_longitude,
           patient_skills, emb_table):
    pid = patient_id.astype(jnp.int32)
    skills_flat = patient_skills.astype(jnp.int32).reshape(-1)
    tt = emb_table.T
    table_lin = _k1(tt, tt, tt, tt).reshape(4 * _S, D)
    flat = _patient_sc(table_lin, pid, patient_age, patient_dialysis_freq,
                       patient_dialysis_latitude,
                       patient_dialysis_longitude, skills_flat)
    return _k3(flat.reshape(B, OUT_P)).T


# speedup vs baseline: 2.6904x; 1.3689x over previous
import functools

import jax
import jax.numpy as jnp
from jax import lax
from jax.experimental import pallas as pl
from jax.experimental.pallas import tpu as pltpu
from jax.experimental.pallas import tpu_sc as plsc

B = 16384
D = 32
NSK = 8
NUM_SKILL_COLS = 65
OUT_D = D + 4 + NUM_SKILL_COLS
OUT_P = 128
V = 100001
NW = 32
BPW = B // NW
AGE_MEAN = 45.0
AGE_STD = 20.0

_K1_C = 896
_NB = 28
_S = _NB * _K1_C


def _k1_body(t0, t1, t2, t3, out):
    x = jnp.concatenate([t0[...], t1[...], t2[...], t3[...]], axis=0)
    out[...] = x.T


_k1 = pl.pallas_call(
    _k1_body,
    grid=(_NB,),
    in_specs=[
        pl.BlockSpec((D, _K1_C), lambda i, k=k: (0, i + k * _NB))
        for k in range(4)
    ],
    out_specs=pl.BlockSpec((_K1_C, OUT_P), lambda i: (i, 0)),
    out_shape=jax.ShapeDtypeStruct((_S, OUT_P), jnp.float32),
)


def _body(table_hbm, pid_hbm, age_hbm, freq_hbm, lat_hbm, lon_hbm,
          skills_hbm, out_hbm, idx_v, rows_v, skills_v, age_v, freq_v,
          lat_v, lon_v, slab, sem):
    c = lax.axis_index("c")
    s = lax.axis_index("s")
    wid = s * 2 + c
    base = wid * BPW

    pltpu.sync_copy(pid_hbm.at[pl.ds(base, BPW)], idx_v)

    one = jnp.ones((16,), jnp.int32)
    nil = jnp.zeros((16,), jnp.int32)

    @plsc.parallel_loop(0, BPW // 16, unroll=4)
    def _remap(i):
        r = idx_v[pl.ds(i * 16, 16)]
        a = (jnp.where(r >= _S, one, nil) + jnp.where(r >= 2 * _S, one, nil)
             + jnp.where(r >= 3 * _S, one, nil))
        idx_v[pl.ds(i * 16, 16)] = (r - a * _S) * 4 + a

    gather = pltpu.async_copy(table_hbm.at[idx_v], rows_v, sem)
    pltpu.sync_copy(skills_hbm.at[pl.ds(wid * 4, 4)], skills_v)
    pltpu.sync_copy(age_hbm.at[pl.ds(base, BPW)], age_v)
    pltpu.sync_copy(freq_hbm.at[pl.ds(base, BPW)], freq_v)
    pltpu.sync_copy(lat_hbm.at[pl.ds(base, BPW)], lat_v)
    pltpu.sync_copy(lon_hbm.at[pl.ds(base, BPW)], lon_v)

    iota = lax.iota(jnp.int32, 16)
    zeros = jnp.zeros((16,), jnp.float32)
    ones = jnp.ones((16,), jnp.float32)

    @plsc.parallel_loop(0, BPW, unroll=8)
    def _zero(r):
        o = r * OUT_P
        slab[pl.ds(o + 36, 16)] = zeros
        slab[pl.ds(o + 52, 16)] = zeros
        slab[pl.ds(o + 68, 16)] = zeros
        slab[pl.ds(o + 84, 16)] = zeros
        slab[pl.ds(o + 85, 16)] = zeros

    iota_row = iota * OUT_P

    @plsc.parallel_loop(0, (BPW * NSK) // 16, unroll=8)
    def _mh(j):
        a = lax.shift_right_logical(j, 6)
        k = lax.bitwise_and(lax.shift_right_logical(j, 3), 7)
        p = lax.bitwise_and(j, 7)
        sk = skills_v[a, k, pl.ds(p * 16, 16)]
        flat = (a * 128 + p * 16) * OUT_P + iota_row + 36 + sk
        plsc.store_scatter(slab, [flat], ones)

    @plsc.parallel_loop(0, BPW // 16, unroll=4)
    def _scal(i):
        b16 = i * 16
        flat = b16 * OUT_P + iota_row + D
        a = (age_v[pl.ds(b16, 16)] - AGE_MEAN) / AGE_STD
        plsc.store_scatter(slab, [flat], a)
        plsc.store_scatter(slab, [flat + 1], freq_v[pl.ds(b16, 16)])
        plsc.store_scatter(slab, [flat + 2], lat_v[pl.ds(b16, 16)])
        plsc.store_scatter(slab, [flat + 3], lon_v[pl.ds(b16, 16)])

    gather.wait()

    @plsc.parallel_loop(0, BPW, unroll=8)
    def _emb(r):
        o = r * OUT_P
        slab[pl.ds(o, 16)] = rows_v[r, pl.ds(0, 16)]
        slab[pl.ds(o + 16, 16)] = rows_v[r, pl.ds(16, 16)]

    pltpu.sync_copy(slab, out_hbm.at[pl.ds(base * OUT_P, BPW * OUT_P)])


_patient_sc = functools.partial(
    pl.kernel,
    out_type=jax.ShapeDtypeStruct((B * OUT_P,), jnp.float32),
    mesh=plsc.VectorSubcoreMesh(core_axis_name="c", subcore_axis_name="s"),
    compiler_params=pltpu.CompilerParams(
        needs_layout_passes=False, use_tc_tiling_on_sc=False),
    scratch_types=[
        pltpu.VMEM((BPW,), jnp.int32),
        pltpu.VMEM((BPW, D), jnp.float32),
        pltpu.VMEM((4, NSK, 128), jnp.int32),
        pltpu.VMEM((BPW,), jnp.float32),
        pltpu.VMEM((BPW,), jnp.float32),
        pltpu.VMEM((BPW,), jnp.float32),
        pltpu.VMEM((BPW,), jnp.float32),
        pltpu.VMEM((BPW * OUT_P,), jnp.float32),
        pltpu.SemaphoreType.DMA,
    ],
)(_body)

_K3_RB = 1024


def _k3_body(fin, out):
    x = fin[...]
    out[...] = x.T[:OUT_D, :]


_k3 = pl.pallas_call(
    _k3_body,
    grid=(B // _K3_RB,),
    in_specs=[pl.BlockSpec((_K3_RB, OUT_P), lambda i: (i, 0))],
    out_specs=pl.BlockSpec((OUT_D, _K3_RB), lambda i: (0, i)),
    out_shape=jax.ShapeDtypeStruct((OUT_D, B), jnp.float32),
)


@jax.jit
def kernel(patient_id, patient_age, patient_dialysis_freq,
           patient_dialysis_latitude, patient_dialysis_longitude,
           patient_skills, emb_table):
    pid = patient_id.astype(jnp.int32)
    skills_n = jnp.transpose(
        patient_skills.astype(jnp.int32).reshape(128, 128, NSK), (0, 2, 1))
    tt = emb_table.T
    table_lin = _k1(tt, tt, tt, tt).reshape(4 * _S, D)
    flat = _patient_sc(table_lin, pid, patient_age, patient_dialysis_freq,
                       patient_dialysis_latitude,
                       patient_dialysis_longitude, skills_n)
    return _k3(flat.reshape(B, OUT_P)).T


# speedup vs baseline: 3.1954x; 1.1877x over previous
import functools

import jax
import jax.numpy as jnp
from jax import lax
from jax.experimental import pallas as pl
from jax.experimental.pallas import tpu as pltpu
from jax.experimental.pallas import tpu_sc as plsc

B = 16384
D = 32
NSK = 8
NUM_SKILL_COLS = 65
OUT_D = D + 4 + NUM_SKILL_COLS
OUT_P = 128
V = 100001
NW = 32
BPW = B // NW
AGE_MEAN = 45.0
AGE_STD = 20.0

_K1_C = 1792
_NB = 14
_S = _NB * _K1_C


def _k1_body(t0, t1, t2, t3, out):
    x = jnp.concatenate([t0[...], t1[...], t2[...], t3[...]], axis=0)
    out[...] = x.T


_k1 = pl.pallas_call(
    _k1_body,
    grid=(_NB,),
    in_specs=[
        pl.BlockSpec((D, _K1_C), lambda i, k=k: (0, i + k * _NB))
        for k in range(4)
    ],
    out_specs=pl.BlockSpec((_K1_C, OUT_P), lambda i: (i, 0)),
    out_shape=jax.ShapeDtypeStruct((_S, OUT_P), jnp.float32),
)


def _body(table_hbm, pid_hbm, age_hbm, freq_hbm, lat_hbm, lon_hbm,
          skills_hbm, out_hbm, idx_v, rows_v, skills_v, age_v, freq_v,
          lat_v, lon_v, slab, sem):
    c = lax.axis_index("c")
    s = lax.axis_index("s")
    wid = s * 2 + c
    base = wid * BPW

    pltpu.sync_copy(pid_hbm.at[pl.ds(base, BPW)], idx_v)

    one = jnp.ones((16,), jnp.int32)
    nil = jnp.zeros((16,), jnp.int32)

    @plsc.parallel_loop(0, BPW // 16, unroll=4)
    def _remap(i):
        r = idx_v[pl.ds(i * 16, 16)]
        a = (jnp.where(r >= _S, one, nil) + jnp.where(r >= 2 * _S, one, nil)
             + jnp.where(r >= 3 * _S, one, nil))
        idx_v[pl.ds(i * 16, 16)] = (r - a * _S) * 4 + a

    gather = pltpu.async_copy(table_hbm.at[idx_v], rows_v, sem)
    pltpu.sync_copy(skills_hbm.at[pl.ds(wid * 4, 4)], skills_v)
    pltpu.sync_copy(age_hbm.at[pl.ds(base, BPW)], age_v)
    pltpu.sync_copy(freq_hbm.at[pl.ds(base, BPW)], freq_v)
    pltpu.sync_copy(lat_hbm.at[pl.ds(base, BPW)], lat_v)
    pltpu.sync_copy(lon_hbm.at[pl.ds(base, BPW)], lon_v)

    iota = lax.iota(jnp.int32, 16)
    zeros = jnp.zeros((16,), jnp.float32)
    ones = jnp.ones((16,), jnp.float32)

    @plsc.parallel_loop(0, BPW, unroll=8)
    def _zero(r):
        o = r * OUT_P
        slab[pl.ds(o + 36, 16)] = zeros
        slab[pl.ds(o + 52, 16)] = zeros
        slab[pl.ds(o + 68, 16)] = zeros
        slab[pl.ds(o + 84, 16)] = zeros
        slab[pl.ds(o + 85, 16)] = zeros

    iota_row = iota * OUT_P

    @plsc.parallel_loop(0, (BPW * NSK) // 16, unroll=8)
    def _mh(j):
        a = lax.shift_right_logical(j, 6)
        k = lax.bitwise_and(lax.shift_right_logical(j, 3), 7)
        p = lax.bitwise_and(j, 7)
        sk = skills_v[a, k, pl.ds(p * 16, 16)]
        flat = (a * 128 + p * 16) * OUT_P + iota_row + 36 + sk
        plsc.store_scatter(slab, [flat], ones)

    @plsc.parallel_loop(0, BPW // 16, unroll=4)
    def _scal(i):
        b16 = i * 16
        flat = b16 * OUT_P + iota_row + D
        a = (age_v[pl.ds(b16, 16)] - AGE_MEAN) / AGE_STD
        plsc.store_scatter(slab, [flat], a)
        plsc.store_scatter(slab, [flat + 1], freq_v[pl.ds(b16, 16)])
        plsc.store_scatter(slab, [flat + 2], lat_v[pl.ds(b16, 16)])
        plsc.store_scatter(slab, [flat + 3], lon_v[pl.ds(b16, 16)])

    gather.wait()

    @plsc.parallel_loop(0, BPW, unroll=8)
    def _emb(r):
        o = r * OUT_P
        slab[pl.ds(o, 16)] = rows_v[r, pl.ds(0, 16)]
        slab[pl.ds(o + 16, 16)] = rows_v[r, pl.ds(16, 16)]

    pltpu.sync_copy(slab, out_hbm.at[pl.ds(base * OUT_P, BPW * OUT_P)])


_patient_sc = functools.partial(
    pl.kernel,
    out_type=jax.ShapeDtypeStruct((B * OUT_P,), jnp.float32),
    mesh=plsc.VectorSubcoreMesh(core_axis_name="c", subcore_axis_name="s"),
    compiler_params=pltpu.CompilerParams(
        needs_layout_passes=False, use_tc_tiling_on_sc=False),
    scratch_types=[
        pltpu.VMEM((BPW,), jnp.int32),
        pltpu.VMEM((BPW, D), jnp.float32),
        pltpu.VMEM((4, NSK, 128), jnp.int32),
        pltpu.VMEM((BPW,), jnp.float32),
        pltpu.VMEM((BPW,), jnp.float32),
        pltpu.VMEM((BPW,), jnp.float32),
        pltpu.VMEM((BPW,), jnp.float32),
        pltpu.VMEM((BPW * OUT_P,), jnp.float32),
        pltpu.SemaphoreType.DMA,
    ],
)(_body)

_K3_RB = 2048


def _k3_body(fin, out):
    x = fin[...]
    out[...] = x.T[:OUT_D, :]


_k3 = pl.pallas_call(
    _k3_body,
    grid=(B // _K3_RB,),
    in_specs=[pl.BlockSpec((_K3_RB, OUT_P), lambda i: (i, 0))],
    out_specs=pl.BlockSpec((OUT_D, _K3_RB), lambda i: (0, i)),
    out_shape=jax.ShapeDtypeStruct((OUT_D, B), jnp.float32),
)


@jax.jit
def kernel(patient_id, patient_age, patient_dialysis_freq,
           patient_dialysis_latitude, patient_dialysis_longitude,
           patient_skills, emb_table):
    pid = patient_id.astype(jnp.int32)
    skills_n = jnp.transpose(
        patient_skills.astype(jnp.int32).reshape(128, 128, NSK), (0, 2, 1))
    tt = emb_table.T
    table_lin = _k1(tt, tt, tt, tt).reshape(4 * _S, D)
    flat = _patient_sc(table_lin, pid, patient_age, patient_dialysis_freq,
                       patient_dialysis_latitude,
                       patient_dialysis_longitude, skills_n)
    return _k3(flat.reshape(B, OUT_P)).T


# speedup vs baseline: 3.6213x; 1.1333x over previous
import functools

import jax
import jax.numpy as jnp
from jax import lax
from jax.experimental import pallas as pl
from jax.experimental.pallas import tpu as pltpu
from jax.experimental.pallas import tpu_sc as plsc

B = 16384
D = 32
NSK = 8
NUM_SKILL_COLS = 65
OUT_D = D + 4 + NUM_SKILL_COLS
OUT_P = 128
V = 100001
NW = 32
BPW = B // NW
AGE_MEAN = 45.0
AGE_STD = 20.0

_K1_C = 3584
_NB = 7
_S = _NB * _K1_C


def _k1_body(t0, t1, t2, t3, out):
    x = jnp.concatenate([t0[...], t1[...], t2[...], t3[...]], axis=0)
    out[...] = x.T


_k1 = pl.pallas_call(
    _k1_body,
    grid=(_NB,),
    in_specs=[
        pl.BlockSpec((D, _K1_C), lambda i, k=k: (0, i + k * _NB))
        for k in range(4)
    ],
    out_specs=pl.BlockSpec((_K1_C, OUT_P), lambda i: (i, 0)),
    out_shape=jax.ShapeDtypeStruct((_S, OUT_P), jnp.float32),
)


def _body(table_hbm, pid_hbm, age_hbm, freq_hbm, lat_hbm, lon_hbm,
          skills_hbm, out_hbm, idx_v, rows_v, skills_v, age_v, freq_v,
          lat_v, lon_v, slab, sem):
    c = lax.axis_index("c")
    s = lax.axis_index("s")
    wid = s * 2 + c
    base = wid * BPW

    pltpu.sync_copy(pid_hbm.at[pl.ds(base, BPW)], idx_v)

    one = jnp.ones((16,), jnp.int32)
    nil = jnp.zeros((16,), jnp.int32)

    @plsc.parallel_loop(0, BPW // 16, unroll=4)
    def _remap(i):
        r = idx_v[pl.ds(i * 16, 16)]
        a = (jnp.where(r >= _S, one, nil) + jnp.where(r >= 2 * _S, one, nil)
             + jnp.where(r >= 3 * _S, one, nil))
        idx_v[pl.ds(i * 16, 16)] = (r - a * _S) * 4 + a

    gather = pltpu.async_copy(table_hbm.at[idx_v], rows_v, sem)
    pltpu.sync_copy(skills_hbm.at[pl.ds(wid * 4, 4)], skills_v)
    pltpu.sync_copy(age_hbm.at[pl.ds(base, BPW)], age_v)
    pltpu.sync_copy(freq_hbm.at[pl.ds(base, BPW)], freq_v)
    pltpu.sync_copy(lat_hbm.at[pl.ds(base, BPW)], lat_v)
    pltpu.sync_copy(lon_hbm.at[pl.ds(base, BPW)], lon_v)

    iota = lax.iota(jnp.int32, 16)
    zeros = jnp.zeros((16,), jnp.float32)
    ones = jnp.ones((16,), jnp.float32)

    @plsc.parallel_loop(0, BPW, unroll=8)
    def _zero(r):
        o = r * OUT_P
        slab[pl.ds(o + 36, 16)] = zeros
        slab[pl.ds(o + 52, 16)] = zeros
        slab[pl.ds(o + 68, 16)] = zeros
        slab[pl.ds(o + 84, 16)] = zeros
        slab[pl.ds(o + 85, 16)] = zeros

    iota_row = iota * OUT_P

    @plsc.parallel_loop(0, (BPW * NSK) // 16, unroll=8)
    def _mh(j):
        a = lax.shift_right_logical(j, 6)
        k = lax.bitwise_and(lax.shift_right_logical(j, 3), 7)
        p = lax.bitwise_and(j, 7)
        sk = skills_v[a, k, pl.ds(p * 16, 16)]
        flat = (a * 128 + p * 16) * OUT_P + iota_row + 36 + sk
        plsc.store_scatter(slab, [flat], ones)

    @plsc.parallel_loop(0, BPW // 16, unroll=4)
    def _scal(i):
        b16 = i * 16
        flat = b16 * OUT_P + iota_row + D
        a = (age_v[pl.ds(b16, 16)] - AGE_MEAN) / AGE_STD
        plsc.store_scatter(slab, [flat], a)
        plsc.store_scatter(slab, [flat + 1], freq_v[pl.ds(b16, 16)])
        plsc.store_scatter(slab, [flat + 2], lat_v[pl.ds(b16, 16)])
        plsc.store_scatter(slab, [flat + 3], lon_v[pl.ds(b16, 16)])

    gather.wait()

    @plsc.parallel_loop(0, BPW, unroll=8)
    def _emb(r):
        o = r * OUT_P
        slab[pl.ds(o, 16)] = rows_v[r, pl.ds(0, 16)]
        slab[pl.ds(o + 16, 16)] = rows_v[r, pl.ds(16, 16)]

    pltpu.sync_copy(slab, out_hbm.at[pl.ds(base * OUT_P, BPW * OUT_P)])


_patient_sc = functools.partial(
    pl.kernel,
    out_type=jax.ShapeDtypeStruct((B * OUT_P,), jnp.float32),
    mesh=plsc.VectorSubcoreMesh(core_axis_name="c", subcore_axis_name="s"),
    compiler_params=pltpu.CompilerParams(
        needs_layout_passes=False, use_tc_tiling_on_sc=False),
    scratch_types=[
        pltpu.VMEM((BPW,), jnp.int32),
        pltpu.VMEM((BPW, D), jnp.float32),
        pltpu.VMEM((4, NSK, 128), jnp.int32),
        pltpu.VMEM((BPW,), jnp.float32),
        pltpu.VMEM((BPW,), jnp.float32),
        pltpu.VMEM((BPW,), jnp.float32),
        pltpu.VMEM((BPW,), jnp.float32),
        pltpu.VMEM((BPW * OUT_P,), jnp.float32),
        pltpu.SemaphoreType.DMA,
    ],
)(_body)

_K3_RB = 4096


def _k3_body(fin, out):
    x = fin[...]
    out[...] = x.T[:OUT_D, :]


_k3 = pl.pallas_call(
    _k3_body,
    grid=(B // _K3_RB,),
    in_specs=[pl.BlockSpec((_K3_RB, OUT_P), lambda i: (i, 0))],
    out_specs=pl.BlockSpec((OUT_D, _K3_RB), lambda i: (0, i)),
    out_shape=jax.ShapeDtypeStruct((OUT_D, B), jnp.float32),
)


@jax.jit
def kernel(patient_id, patient_age, patient_dialysis_freq,
           patient_dialysis_latitude, patient_dialysis_longitude,
           patient_skills, emb_table):
    pid = patient_id.astype(jnp.int32)
    skills_n = jnp.transpose(
        patient_skills.astype(jnp.int32).reshape(128, 128, NSK), (0, 2, 1))
    tt = emb_table.T
    table_lin = _k1(tt, tt, tt, tt).reshape(4 * _S, D)
    flat = _patient_sc(table_lin, pid, patient_age, patient_dialysis_freq,
                       patient_dialysis_latitude,
                       patient_dialysis_longitude, skills_n)
    return _k3(flat.reshape(B, OUT_P)).T


# speedup vs baseline: 3.8007x; 1.0496x over previous
import functools

import jax
import jax.numpy as jnp
from jax import lax
from jax.experimental import pallas as pl
from jax.experimental.pallas import tpu as pltpu
from jax.experimental.pallas import tpu_sc as plsc

B = 16384
D = 32
NSK = 8
NUM_SKILL_COLS = 65
OUT_D = D + 4 + NUM_SKILL_COLS
OUT_P = 128
V = 100001
NW = 32
BPW = B // NW
AGE_MEAN = 45.0
AGE_STD = 20.0

_K1_C = 6272
_NB = 4
_S = _NB * _K1_C


def _k1_body(t0, t1, t2, t3, out):
    x = jnp.concatenate([t0[...], t1[...], t2[...], t3[...]], axis=0)
    out[...] = x.T


_k1 = pl.pallas_call(
    _k1_body,
    grid=(_NB,),
    in_specs=[
        pl.BlockSpec((D, _K1_C), lambda i, k=k: (0, i + k * _NB))
        for k in range(4)
    ],
    out_specs=pl.BlockSpec((_K1_C, OUT_P), lambda i: (i, 0)),
    out_shape=jax.ShapeDtypeStruct((_S, OUT_P), jnp.float32),
)


def _body(table_hbm, pid_hbm, age_hbm, freq_hbm, lat_hbm, lon_hbm,
          skills_hbm, out_hbm, idx_v, rows_v, skills_v, age_v, freq_v,
          lat_v, lon_v, slab, sem, sem2):
    c = lax.axis_index("c")
    s = lax.axis_index("s")
    wid = s * 2 + c
    base = wid * BPW

    pltpu.sync_copy(pid_hbm.at[pl.ds(base, BPW)], idx_v)

    one = jnp.ones((16,), jnp.int32)
    nil = jnp.zeros((16,), jnp.int32)

    @plsc.parallel_loop(0, BPW // 16, unroll=4)
    def _remap(i):
        r = idx_v[pl.ds(i * 16, 16)]
        a = (jnp.where(r >= _S, one, nil) + jnp.where(r >= 2 * _S, one, nil)
             + jnp.where(r >= 3 * _S, one, nil))
        idx_v[pl.ds(i * 16, 16)] = (r - a * _S) * 4 + a

    gather = pltpu.async_copy(table_hbm.at[idx_v], rows_v, sem)
    pltpu.sync_copy(skills_hbm.at[pl.ds(wid * 4, 4)], skills_v)
    pltpu.sync_copy(age_hbm.at[pl.ds(base, BPW)], age_v)
    pltpu.sync_copy(freq_hbm.at[pl.ds(base, BPW)], freq_v)
    pltpu.sync_copy(lat_hbm.at[pl.ds(base, BPW)], lat_v)
    pltpu.sync_copy(lon_hbm.at[pl.ds(base, BPW)], lon_v)

    iota = lax.iota(jnp.int32, 16)
    zeros = jnp.zeros((16,), jnp.float32)
    ones = jnp.ones((16,), jnp.float32)
    iota_row = iota * OUT_P

    gather.wait()

    handles = []
    for ch in range(4):
        r0 = ch * 128

        @plsc.parallel_loop(0, 128, unroll=8)
        def _zero(r, r0=r0):
            o = (r0 + r) * OUT_P
            slab[pl.ds(o + 36, 16)] = zeros
            slab[pl.ds(o + 52, 16)] = zeros
            slab[pl.ds(o + 68, 16)] = zeros
            slab[pl.ds(o + 84, 16)] = zeros
            slab[pl.ds(o + 85, 16)] = zeros

        @plsc.parallel_loop(0, 128, unroll=8)
        def _emb(r, r0=r0):
            o = (r0 + r) * OUT_P
            slab[pl.ds(o, 16)] = rows_v[r0 + r, pl.ds(0, 16)]
            slab[pl.ds(o + 16, 16)] = rows_v[r0 + r, pl.ds(16, 16)]

        @plsc.parallel_loop(0, NSK * NSK, unroll=8)
        def _mh(j, ch=ch, r0=r0):
            k = lax.shift_right_logical(j, 3)
            p = lax.bitwise_and(j, 7)
            sk = skills_v[ch, k, pl.ds(p * 16, 16)]
            flat = (r0 + p * 16) * OUT_P + iota_row + 36 + sk
            plsc.store_scatter(slab, [flat], ones)

        @plsc.parallel_loop(0, 8, unroll=4)
        def _scal(i, r0=r0):
            b16 = r0 + i * 16
            flat = b16 * OUT_P + iota_row + D
            a = (age_v[pl.ds(b16, 16)] - AGE_MEAN) / AGE_STD
            plsc.store_scatter(slab, [flat], a)
            plsc.store_scatter(slab, [flat + 1], freq_v[pl.ds(b16, 16)])
            plsc.store_scatter(slab, [flat + 2], lat_v[pl.ds(b16, 16)])
            plsc.store_scatter(slab, [flat + 3], lon_v[pl.ds(b16, 16)])

        handles.append(pltpu.async_copy(
            slab.at[pl.ds(r0 * OUT_P, 128 * OUT_P)],
            out_hbm.at[pl.ds((base + r0) * OUT_P, 128 * OUT_P)], sem2))

    for h in handles:
        h.wait()


_patient_sc = functools.partial(
    pl.kernel,
    out_type=jax.ShapeDtypeStruct((B * OUT_P,), jnp.float32),
    mesh=plsc.VectorSubcoreMesh(core_axis_name="c", subcore_axis_name="s"),
    compiler_params=pltpu.CompilerParams(
        needs_layout_passes=False, use_tc_tiling_on_sc=False),
    scratch_types=[
        pltpu.VMEM((BPW,), jnp.int32),
        pltpu.VMEM((BPW, D), jnp.float32),
        pltpu.VMEM((4, NSK, 128), jnp.int32),
        pltpu.VMEM((BPW,), jnp.float32),
        pltpu.VMEM((BPW,), jnp.float32),
        pltpu.VMEM((BPW,), jnp.float32),
        pltpu.VMEM((BPW,), jnp.float32),
        pltpu.VMEM((BPW * OUT_P,), jnp.float32),
        pltpu.SemaphoreType.DMA,
        pltpu.SemaphoreType.DMA,
    ],
)(_body)

_K3_RB = 4096


def _k3_body(fin, out):
    x = fin[...]
    out[...] = x.T[:OUT_D, :]


_k3 = pl.pallas_call(
    _k3_body,
    grid=(B // _K3_RB,),
    in_specs=[pl.BlockSpec((_K3_RB, OUT_P), lambda i: (i, 0))],
    out_specs=pl.BlockSpec((OUT_D, _K3_RB), lambda i: (0, i)),
    out_shape=jax.ShapeDtypeStruct((OUT_D, B), jnp.float32),
)


@jax.jit
def kernel(patient_id, patient_age, patient_dialysis_freq,
           patient_dialysis_latitude, patient_dialysis_longitude,
           patient_skills, emb_table):
    pid = patient_id.astype(jnp.int32)
    skills_n = jnp.transpose(
        patient_skills.astype(jnp.int32).reshape(128, 128, NSK), (0, 2, 1))
    tt = emb_table.T
    table_lin = _k1(tt, tt, tt, tt).reshape(4 * _S, D)
    flat = _patient_sc(table_lin, pid, patient_age, patient_dialysis_freq,
                       patient_dialysis_latitude,
                       patient_dialysis_longitude, skills_n)
    return _k3(flat.reshape(B, OUT_P)).T


# speedup vs baseline: 3.8308x; 1.0079x over previous
import functools

import jax
import jax.numpy as jnp
from jax import lax
from jax.experimental import pallas as pl
from jax.experimental.pallas import tpu as pltpu
from jax.experimental.pallas import tpu_sc as plsc

B = 16384
D = 32
NSK = 8
NUM_SKILL_COLS = 65
OUT_D = D + 4 + NUM_SKILL_COLS
OUT_P = 128
V = 100001
NW = 32
BPW = B // NW
AGE_MEAN = 45.0
AGE_STD = 20.0

_K1_C = 6272
_NB = 4
_S = _NB * _K1_C


def _k1_body(t0, t1, t2, t3, out):
    x = jnp.concatenate([t0[...], t1[...], t2[...], t3[...]], axis=0)
    out[...] = x.T


_k1 = pl.pallas_call(
    _k1_body,
    grid=(_NB,),
    in_specs=[
        pl.BlockSpec((D, _K1_C), lambda i, k=k: (0, i + k * _NB))
        for k in range(4)
    ],
    out_specs=pl.BlockSpec((_K1_C, OUT_P), lambda i: (i, 0)),
    out_shape=jax.ShapeDtypeStruct((_S, OUT_P), jnp.float32),
)


def _body(table_hbm, pid_hbm, age_hbm, freq_hbm, lat_hbm, lon_hbm,
          skills_hbm, out_hbm, idx_v, rows_v, skills_v, age_v, freq_v,
          lat_v, lon_v, slab, sem, sem2):
    c = lax.axis_index("c")
    s = lax.axis_index("s")
    wid = s * 2 + c
    base = wid * BPW

    pltpu.sync_copy(pid_hbm.at[pl.ds(base, BPW)], idx_v)

    one = jnp.ones((16,), jnp.int32)
    nil = jnp.zeros((16,), jnp.int32)

    @plsc.parallel_loop(0, BPW // 16, unroll=4)
    def _remap(i):
        r = idx_v[pl.ds(i * 16, 16)]
        a = (jnp.where(r >= _S, one, nil) + jnp.where(r >= 2 * _S, one, nil)
             + jnp.where(r >= 3 * _S, one, nil))
        idx_v[pl.ds(i * 16, 16)] = (r - a * _S) * 4 + a

    gathers = [
        pltpu.async_copy(table_hbm.at[idx_v.at[pl.ds(g * 128, 128)]],
                         rows_v.at[pl.ds(g * 128, 128)], sem)
        for g in range(4)
    ]
    pltpu.sync_copy(skills_hbm.at[pl.ds(wid * 4, 4)], skills_v)
    pltpu.sync_copy(age_hbm.at[pl.ds(base, BPW)], age_v)
    pltpu.sync_copy(freq_hbm.at[pl.ds(base, BPW)], freq_v)
    pltpu.sync_copy(lat_hbm.at[pl.ds(base, BPW)], lat_v)
    pltpu.sync_copy(lon_hbm.at[pl.ds(base, BPW)], lon_v)

    iota = lax.iota(jnp.int32, 16)
    zeros = jnp.zeros((16,), jnp.float32)
    ones = jnp.ones((16,), jnp.float32)
    iota_row = iota * OUT_P

    handles = []
    for ch in range(4):
        r0 = ch * 128

        @plsc.parallel_loop(0, 128, unroll=8)
        def _zero(r, r0=r0):
            o = (r0 + r) * OUT_P
            slab[pl.ds(o + 36, 16)] = zeros
            slab[pl.ds(o + 52, 16)] = zeros
            slab[pl.ds(o + 68, 16)] = zeros
            slab[pl.ds(o + 84, 16)] = zeros
            slab[pl.ds(o + 85, 16)] = zeros

        @plsc.parallel_loop(0, NSK * NSK, unroll=8)
        def _mh(j, ch=ch, r0=r0):
            k = lax.shift_right_logical(j, 3)
            p = lax.bitwise_and(j, 7)
            sk = skills_v[ch, k, pl.ds(p * 16, 16)]
            flat = (r0 + p * 16) * OUT_P + iota_row + 36 + sk
            plsc.store_scatter(slab, [flat], ones)

        @plsc.parallel_loop(0, 8, unroll=4)
        def _scal(i, r0=r0):
            b16 = r0 + i * 16
            flat = b16 * OUT_P + iota_row + D
            a = (age_v[pl.ds(b16, 16)] - AGE_MEAN) / AGE_STD
            plsc.store_scatter(slab, [flat], a)
            plsc.store_scatter(slab, [flat + 1], freq_v[pl.ds(b16, 16)])
            plsc.store_scatter(slab, [flat + 2], lat_v[pl.ds(b16, 16)])
            plsc.store_scatter(slab, [flat + 3], lon_v[pl.ds(b16, 16)])

        gathers[ch].wait()

        @plsc.parallel_loop(0, 128, unroll=8)
        def _emb(r, r0=r0):
            o = (r0 + r) * OUT_P
            slab[pl.ds(o, 16)] = rows_v[r0 + r, pl.ds(0, 16)]
            slab[pl.ds(o + 16, 16)] = rows_v[r0 + r, pl.ds(16, 16)]

        handles.append(pltpu.async_copy(
            slab.at[pl.ds(r0 * OUT_P, 128 * OUT_P)],
            out_hbm.at[pl.ds((base + r0) * OUT_P, 128 * OUT_P)], sem2))

    for h in handles:
        h.wait()


_patient_sc = functools.partial(
    pl.kernel,
    out_type=jax.ShapeDtypeStruct((B * OUT_P,), jnp.float32),
    mesh=plsc.VectorSubcoreMesh(core_axis_name="c", subcore_axis_name="s"),
    compiler_params=pltpu.CompilerParams(
        needs_layout_passes=False, use_tc_tiling_on_sc=False),
    scratch_types=[
        pltpu.VMEM((BPW,), jnp.int32),
        pltpu.VMEM((BPW, D), jnp.float32),
        pltpu.VMEM((4, NSK, 128), jnp.int32),
        pltpu.VMEM((BPW,), jnp.float32),
        pltpu.VMEM((BPW,), jnp.float32),
        pltpu.VMEM((BPW,), jnp.float32),
        pltpu.VMEM((BPW,), jnp.float32),
        pltpu.VMEM((BPW * OUT_P,), jnp.float32),
        pltpu.SemaphoreType.DMA,
        pltpu.SemaphoreType.DMA,
    ],
)(_body)

_K3_RB = 4096


def _k3_body(fin, out):
    x = fin[...]
    out[...] = x.T[:OUT_D, :]


_k3 = pl.pallas_call(
    _k3_body,
    grid=(B // _K3_RB,),
    in_specs=[pl.BlockSpec((_K3_RB, OUT_P), lambda i: (i, 0))],
    out_specs=pl.BlockSpec((OUT_D, _K3_RB), lambda i: (0, i)),
    out_shape=jax.ShapeDtypeStruct((OUT_D, B), jnp.float32),
)


@jax.jit
def kernel(patient_id, patient_age, patient_dialysis_freq,
           patient_dialysis_latitude, patient_dialysis_longitude,
           patient_skills, emb_table):
    pid = patient_id.astype(jnp.int32)
    skills_n = jnp.transpose(
        patient_skills.astype(jnp.int32).reshape(128, 128, NSK), (0, 2, 1))
    tt = emb_table.T
    table_lin = _k1(tt, tt, tt, tt).reshape(4 * _S, D)
    flat = _patient_sc(table_lin, pid, patient_age, patient_dialysis_freq,
                       patient_dialysis_latitude,
                       patient_dialysis_longitude, skills_n)
    return _k3(flat.reshape(B, OUT_P)).T
